# baseline (speedup 1.0000x reference)
import functools

import jax
import jax.numpy as jnp
from jax import lax
from jax.experimental import pallas as pl
from jax.experimental.pallas import tpu as pltpu
from jax.experimental.pallas import tpu_sc as plsc

B, S, H = 2, 2048, 2048
F = 4096
E = 8
T = B * S
TILE = 512
G = 4
GTILE = G * TILE
NG = 11
P = NG * GTILE
FB = 256
NF = F // FB

NC, NS = 2, 16
NW = NC * NS
TPW = T // NW


def _route_body(hid_ref, wqkv_ref, dests_ref, probs_ref, meta_ref):
    hid = hid_ref[...]
    wqkv = wqkv_ref[...]
    mixed = lax.dot_general(wqkv, hid, (((1,), (1,)), ((), ())),
                            preferred_element_type=jnp.float32)
    q = mixed[0:E, :]
    k = mixed[E:2 * E, :]
    v = mixed[2 * E:3 * E, :]
    cols = []
    for i in range(E):
        s = q[i:i + 1, :] * k
        m = jnp.max(s, axis=0, keepdims=True)
        ex = jnp.exp(s - m)
        attn = ex / jnp.sum(ex, axis=0, keepdims=True)
        cols.append(jnp.sum(attn * v, axis=0, keepdims=True))
    logits = jnp.concatenate(cols, axis=0)

    lane = lax.broadcasted_iota(jnp.int32, (E, T), 0)
    m1 = jnp.max(logits, axis=0, keepdims=True)
    i1 = jnp.min(jnp.where(logits >= m1, lane, E), axis=0, keepdims=True)
    masked = jnp.where(lane == i1, jnp.float32(-1e30), logits)
    m2 = jnp.max(masked, axis=0, keepdims=True)
    i2 = jnp.min(jnp.where(masked >= m2, lane, E), axis=0, keepdims=True)
    r = jnp.exp(m2 - m1)
    p1 = 1.0 / (1.0 + r)
    p2 = r / (1.0 + r)

    oh1 = (lane == i1).astype(jnp.float32)
    oh2 = (lane == i2).astype(jnp.float32)
    c = oh1 + oh2
    incl = c
    d = 1
    while d < T:
        incl = incl + jnp.concatenate(
            [jnp.zeros((E, d), jnp.float32), incl[:, :T - d]], axis=1)
        d *= 2
    excl = (incl - c).astype(jnp.int32)
    counts = incl[:, T - 1:T].astype(jnp.int32)
    st_cnt = (counts + (TILE - 1)) // TILE
    grp_cnt = (counts + (GTILE - 1)) // GTILE
    incl_g = grp_cnt
    d = 1
    while d < E:
        incl_g = incl_g + jnp.concatenate(
            [jnp.zeros((d, 1), jnp.int32), incl_g[:E - d, :]], axis=0)
        d *= 2
    excl_g = incl_g - grp_cnt
    row_off = excl_g * GTILE
    nactg = incl_g[E - 1:E, :]

    dest_base = excl + row_off
    dest1 = jnp.sum(jnp.where(lane == i1, dest_base, 0), axis=0, keepdims=True)
    dest2 = jnp.sum(jnp.where(lane == i2, dest_base, 0), axis=0, keepdims=True)
    dests_ref[0:1, :] = dest1
    dests_ref[1:2, :] = dest2
    probs_ref[0:1, :] = p1
    probs_ref[1:2, :] = p2

    gj = lax.broadcasted_iota(jnp.int32, (E, 64), 1)
    te1 = jnp.minimum(
        jnp.sum((gj >= incl_g).astype(jnp.int32), axis=0, keepdims=True),
        E - 1)
    gj2 = gj - NG
    te2 = jnp.minimum(
        jnp.sum((gj2 >= incl_g).astype(jnp.int32), axis=0, keepdims=True),
        E - 1)
    eidx = lax.broadcasted_iota(jnp.int32, (E, 64), 0)
    oh2 = (eidx == te2).astype(jnp.int32)
    excl_g_j = jnp.sum(oh2 * excl_g, axis=0, keepdims=True)
    grp_j = jnp.sum(oh2 * grp_cnt, axis=0, keepdims=True)
    st_j = jnp.sum(oh2 * st_cnt, axis=0, keepdims=True)
    lane64 = lax.broadcasted_iota(jnp.int32, (1, 64), 1)
    local = (lane64 - NG) - excl_g_j
    stc_j = jnp.where(local < grp_j - 1, G, st_j - G * (grp_j - 1))
    meta_ref[...] = jnp.where(
        lane64 < NG, te1,
        jnp.where(lane64 < 2 * NG, stc_j, nactg))


_route = pl.pallas_call(
    _route_body,
    out_shape=(
        jax.ShapeDtypeStruct((2, T), jnp.int32),
        jax.ShapeDtypeStruct((2, T), jnp.float32),
        jax.ShapeDtypeStruct((1, 64), jnp.int32),
    ),
)


DCH = 32
HW = H // 2


def _dispatch_body(hid_hbm, d1_hbm, d2_hbm, xpad_hbm, i1_v, i2_v, rows_v, sem):
    wid = lax.axis_index("s") * NC + lax.axis_index("c")
    base = wid * TPW

    def body(j, carry):
        tok = base + j * DCH
        pltpu.sync_copy(d1_hbm.at[pl.ds(tok, DCH)], i1_v)
        pltpu.sync_copy(d2_hbm.at[pl.ds(tok, DCH)], i2_v)
        pltpu.sync_copy(hid_hbm.at[pl.ds(tok, DCH)], rows_v)
        pltpu.async_copy(rows_v, xpad_hbm.at[i1_v], sem).wait()
        pltpu.async_copy(rows_v, xpad_hbm.at[i2_v], sem).wait()
        return carry

    lax.fori_loop(0, TPW // DCH, body, 0)


@functools.cache
def _make_dispatch():
    return pl.kernel(
        _dispatch_body,
        mesh=plsc.VectorSubcoreMesh(core_axis_name="c", subcore_axis_name="s"),
        out_type=jax.ShapeDtypeStruct((P, HW), jnp.int32),
        scratch_types=[
            pltpu.VMEM((DCH,), jnp.int32),
            pltpu.VMEM((DCH,), jnp.int32),
            pltpu.VMEM((DCH, HW), jnp.int32),
            pltpu.SemaphoreType.DMA,
        ],
    )


def _gemm_body(meta_ref, x_ref, w1a_ref, w1b_ref, w2_ref, y_ref):
    g = pl.program_id(0)
    f = pl.program_id(1)
    nactg = meta_ref[2 * NG]
    stc = meta_ref[NG + g]

    @pl.when(g < nactg)
    def _():
        w1a = w1a_ref[0].astype(jnp.bfloat16)
        w1b = w1b_ref[0].astype(jnp.bfloat16)
        w2 = w2_ref[0].astype(jnp.bfloat16)

        def half(lo):
            x = x_ref[lo:lo + TILE, :]
            a = lax.dot_general(x, w1a, (((1,), (1,)), ((), ())),
                                preferred_element_type=jnp.float32)
            bg = lax.dot_general(x, w1b, (((1,), (1,)), ((), ())),
                                 preferred_element_type=jnp.float32)
            inter = (a * lax.logistic(a) * bg).astype(jnp.bfloat16)
            y_part = lax.dot_general(inter, w2,
                                     (((1,), (1,)), ((), ())),
                                     preferred_element_type=jnp.float32)
            prev = jnp.where(f == 0, jnp.zeros_like(y_part),
                             y_ref[lo:lo + TILE, :])
            y_ref[lo:lo + TILE, :] = prev + y_part

        half(0)
        for si in range(1, G):
            @pl.when(stc > si)
            def _(lo=si * TILE):
                half(lo)


def _x_map(g, f, m):
    return (jnp.minimum(g, m[2 * NG] - 1), 0)


def _w1a_map(g, f, m):
    return (m[g], jnp.where(g < m[2 * NG], f, NF - 1), 0)


def _w1b_map(g, f, m):
    return (m[g], NF + jnp.where(g < m[2 * NG], f, NF - 1), 0)


def _w2_map(g, f, m):
    return (m[g], 0, jnp.where(g < m[2 * NG], f, NF - 1))


_gemm = pl.pallas_call(
    _gemm_body,
    grid_spec=pltpu.PrefetchScalarGridSpec(
        num_scalar_prefetch=1,
        grid=(NG, NF),
        in_specs=[
            pl.BlockSpec((GTILE, H), _x_map),
            pl.BlockSpec((1, FB, H), _w1a_map),
            pl.BlockSpec((1, FB, H), _w1b_map),
            pl.BlockSpec((1, H, FB), _w2_map),
        ],
        out_specs=pl.BlockSpec((GTILE, H), lambda g, f, m: (g, 0)),
    ),
    out_shape=jax.ShapeDtypeStruct((P, H), jnp.float32),
    compiler_params=pltpu.CompilerParams(
        dimension_semantics=("arbitrary", "arbitrary"),
        vmem_limit_bytes=64 * 1024 * 1024),
)


CCH = 16


def _combine_body(y_hbm, d1_hbm, d2_hbm, p1_hbm, p2_hbm, out_hbm,
                  i1_v, i2_v, p1_v, p2_v, y1_v, y2_v, sem):
    wid = lax.axis_index("s") * NC + lax.axis_index("c")
    base = wid * TPW

    def chunk(j, carry):
        tok = base + j * CCH
        pltpu.sync_copy(d1_hbm.at[pl.ds(tok, CCH)], i1_v)
        pltpu.sync_copy(d2_hbm.at[pl.ds(tok, CCH)], i2_v)
        pltpu.sync_copy(p1_hbm.at[pl.ds(tok, CCH)], p1_v)
        pltpu.sync_copy(p2_hbm.at[pl.ds(tok, CCH)], p2_v)
        cp1 = pltpu.async_copy(y_hbm.at[i1_v], y1_v, sem)
        cp2 = pltpu.async_copy(y_hbm.at[i2_v], y2_v, sem)
        cp1.wait()
        cp2.wait()

        def row(rr, carry2):
            s1 = p1_v[rr]
            s2 = p2_v[rr]
            for cc in range(H // 16):
                a = y1_v[rr, pl.ds(cc * 16, 16)]
                b = y2_v[rr, pl.ds(cc * 16, 16)]
                y1_v[rr, pl.ds(cc * 16, 16)] = s1 * a + s2 * b
            return carry2

        lax.fori_loop(0, CCH, row, 0)
        pltpu.sync_copy(y1_v, out_hbm.at[pl.ds(tok, CCH)])
        return carry

    lax.fori_loop(0, TPW // CCH, chunk, 0)


@functools.cache
def _make_combine():
    return pl.kernel(
        _combine_body,
        mesh=plsc.VectorSubcoreMesh(core_axis_name="c", subcore_axis_name="s"),
        out_type=jax.ShapeDtypeStruct((T, H), jnp.float32),
        scratch_types=[
            pltpu.VMEM((CCH,), jnp.int32),
            pltpu.VMEM((CCH,), jnp.int32),
            pltpu.VMEM((CCH, 16), jnp.float32),
            pltpu.VMEM((CCH, 16), jnp.float32),
            pltpu.VMEM((CCH, H), jnp.float32),
            pltpu.VMEM((CCH, H), jnp.float32),
            pltpu.SemaphoreType.DMA,
        ],
    )


def kernel(hidden_states, Wqkv, W1, W2):
    hid = hidden_states.reshape(T, H)
    dests, probs, meta = _route(hid, Wqkv)
    d1, d2 = dests[0], dests[1]
    p1, p2 = probs[0], probs[1]
    meta_vec = meta[0]
    p1b = jnp.broadcast_to(p1[:, None], (T, 16))
    p2b = jnp.broadcast_to(p2[:, None], (T, 16))
    hid_i32 = lax.bitcast_convert_type(
        hid.astype(jnp.bfloat16).reshape(T, HW, 2), jnp.int32)
    x_pad_i32 = _make_dispatch()(hid_i32, d1, d2)
    x_pad = lax.bitcast_convert_type(x_pad_i32, jnp.bfloat16).reshape(P, H)
    y = _gemm(meta_vec, x_pad, W1, W1, W2)
    out = _make_combine()(y, d1, d2, p1b, p2b)
    return out.reshape(B, S, H)


# speedup vs baseline: 2.1249x; 2.1249x over previous
import functools

import jax
import jax.numpy as jnp
from jax import lax
from jax.experimental import pallas as pl
from jax.experimental.pallas import tpu as pltpu
from jax.experimental.pallas import tpu_sc as plsc

B, S, H = 2, 2048, 2048
F = 4096
E = 8
T = B * S
TILE = 512
G = 3
GTILE = G * TILE
NG = 12
P = NG * GTILE
FB = 256
NF = F // FB

NC, NS = 2, 16
NW = NC * NS
TPW = T // NW


def _route_body(hid_ref, wqkv_ref, dests_ref, probs_ref, meta_ref):
    hid = hid_ref[...]
    wqkv = wqkv_ref[...]
    mixed = lax.dot_general(wqkv, hid, (((1,), (1,)), ((), ())),
                            preferred_element_type=jnp.float32)
    q = mixed[0:E, :]
    k = mixed[E:2 * E, :]
    v = mixed[2 * E:3 * E, :]
    cols = []
    for i in range(E):
        s = q[i:i + 1, :] * k
        m = jnp.max(s, axis=0, keepdims=True)
        ex = jnp.exp(s - m)
        attn = ex / jnp.sum(ex, axis=0, keepdims=True)
        cols.append(jnp.sum(attn * v, axis=0, keepdims=True))
    logits = jnp.concatenate(cols, axis=0)

    lane = lax.broadcasted_iota(jnp.int32, (E, T), 0)
    m1 = jnp.max(logits, axis=0, keepdims=True)
    i1 = jnp.min(jnp.where(logits >= m1, lane, E), axis=0, keepdims=True)
    masked = jnp.where(lane == i1, jnp.float32(-1e30), logits)
    m2 = jnp.max(masked, axis=0, keepdims=True)
    i2 = jnp.min(jnp.where(masked >= m2, lane, E), axis=0, keepdims=True)
    r = jnp.exp(m2 - m1)
    p1 = 1.0 / (1.0 + r)
    p2 = r / (1.0 + r)

    oh1 = (lane == i1).astype(jnp.float32)
    oh2 = (lane == i2).astype(jnp.float32)
    c = oh1 + oh2
    incl = c
    d = 1
    while d < T:
        incl = incl + jnp.concatenate(
            [jnp.zeros((E, d), jnp.float32), incl[:, :T - d]], axis=1)
        d *= 2
    excl = (incl - c).astype(jnp.int32)
    counts = incl[:, T - 1:T].astype(jnp.int32)
    st_cnt = (counts + (TILE - 1)) // TILE
    grp_cnt = (counts + (GTILE - 1)) // GTILE
    incl_g = grp_cnt
    d = 1
    while d < E:
        incl_g = incl_g + jnp.concatenate(
            [jnp.zeros((d, 1), jnp.int32), incl_g[:E - d, :]], axis=0)
        d *= 2
    excl_g = incl_g - grp_cnt
    row_off = excl_g * GTILE
    nactg = incl_g[E - 1:E, :]

    dest_base = excl + row_off
    dest1 = jnp.sum(jnp.where(lane == i1, dest_base, 0), axis=0, keepdims=True)
    dest2 = jnp.sum(jnp.where(lane == i2, dest_base, 0), axis=0, keepdims=True)
    dests_ref[0:1, :] = dest1
    dests_ref[1:2, :] = dest2
    probs_ref[0:1, :] = p1
    probs_ref[1:2, :] = p2

    gj = lax.broadcasted_iota(jnp.int32, (E, 64), 1)
    te1 = jnp.minimum(
        jnp.sum((gj >= incl_g).astype(jnp.int32), axis=0, keepdims=True),
        E - 1)
    gj2 = gj - NG
    te2 = jnp.minimum(
        jnp.sum((gj2 >= incl_g).astype(jnp.int32), axis=0, keepdims=True),
        E - 1)
    eidx = lax.broadcasted_iota(jnp.int32, (E, 64), 0)
    oh2 = (eidx == te2).astype(jnp.int32)
    excl_g_j = jnp.sum(oh2 * excl_g, axis=0, keepdims=True)
    grp_j = jnp.sum(oh2 * grp_cnt, axis=0, keepdims=True)
    st_j = jnp.sum(oh2 * st_cnt, axis=0, keepdims=True)
    lane64 = lax.broadcasted_iota(jnp.int32, (1, 64), 1)
    local = (lane64 - NG) - excl_g_j
    stc_j = jnp.where(local < grp_j - 1, G, st_j - G * (grp_j - 1))
    meta_ref[...] = jnp.where(
        lane64 < NG, te1,
        jnp.where(lane64 < 2 * NG, stc_j, nactg))


_route = pl.pallas_call(
    _route_body,
    out_shape=(
        jax.ShapeDtypeStruct((2, T), jnp.int32),
        jax.ShapeDtypeStruct((2, T), jnp.float32),
        jax.ShapeDtypeStruct((1, 64), jnp.int32),
    ),
)


DCH = 32
HW = H // 2


def _dispatch_body(hid_hbm, d1_hbm, d2_hbm, xpad_hbm, i1_v, i2_v, rows_v, sem):
    wid = lax.axis_index("s") * NC + lax.axis_index("c")
    base = wid * TPW

    def body(j, carry):
        tok = base + j * DCH
        pltpu.sync_copy(d1_hbm.at[pl.ds(tok, DCH)], i1_v)
        pltpu.sync_copy(d2_hbm.at[pl.ds(tok, DCH)], i2_v)
        pltpu.sync_copy(hid_hbm.at[pl.ds(tok, DCH)], rows_v)
        pltpu.async_copy(rows_v, xpad_hbm.at[i1_v], sem).wait()
        pltpu.async_copy(rows_v, xpad_hbm.at[i2_v], sem).wait()
        return carry

    lax.fori_loop(0, TPW // DCH, body, 0)


@functools.cache
def _make_dispatch():
    return pl.kernel(
        _dispatch_body,
        mesh=plsc.VectorSubcoreMesh(core_axis_name="c", subcore_axis_name="s"),
        out_type=jax.ShapeDtypeStruct((P, H), jnp.float32),
        scratch_types=[
            pltpu.VMEM((DCH,), jnp.int32),
            pltpu.VMEM((DCH,), jnp.int32),
            pltpu.VMEM((DCH, H), jnp.float32),
            pltpu.SemaphoreType.DMA,
        ],
    )


def _gemm_body(meta_ref, x_ref, w1a_ref, w1b_ref, w2_ref, y_ref):
    g = pl.program_id(0)
    f = pl.program_id(1)
    nactg = meta_ref[2 * NG]
    stc = meta_ref[NG + g]

    @pl.when(g < nactg)
    def _():
        def half(lo):
            x = x_ref[lo:lo + TILE, :]
            a = lax.dot_general(x, w1a_ref[0], (((1,), (1,)), ((), ())),
                                preferred_element_type=jnp.float32)
            bg = lax.dot_general(x, w1b_ref[0], (((1,), (1,)), ((), ())),
                                 preferred_element_type=jnp.float32)
            inter = a * lax.logistic(a) * bg
            y_part = lax.dot_general(inter, w2_ref[0],
                                     (((1,), (1,)), ((), ())),
                                     preferred_element_type=jnp.float32)
            prev = jnp.where(f == 0, jnp.zeros_like(y_part),
                             y_ref[lo:lo + TILE, :])
            y_ref[lo:lo + TILE, :] = prev + y_part

        half(0)
        for si in range(1, G):
            @pl.when(stc > si)
            def _(lo=si * TILE):
                half(lo)


def _x_map(g, f, m):
    return (jnp.minimum(g, m[2 * NG] - 1), 0)


def _w1a_map(g, f, m):
    return (m[g], jnp.where(g < m[2 * NG], f, NF - 1), 0)


def _w1b_map(g, f, m):
    return (m[g], NF + jnp.where(g < m[2 * NG], f, NF - 1), 0)


def _w2_map(g, f, m):
    return (m[g], 0, jnp.where(g < m[2 * NG], f, NF - 1))


_gemm = pl.pallas_call(
    _gemm_body,
    grid_spec=pltpu.PrefetchScalarGridSpec(
        num_scalar_prefetch=1,
        grid=(NG, NF),
        in_specs=[
            pl.BlockSpec((GTILE, H), _x_map),
            pl.BlockSpec((1, FB, H), _w1a_map),
            pl.BlockSpec((1, FB, H), _w1b_map),
            pl.BlockSpec((1, H, FB), _w2_map),
        ],
        out_specs=pl.BlockSpec((GTILE, H), lambda g, f, m: (g, 0)),
    ),
    out_shape=jax.ShapeDtypeStruct((P, H), jnp.float32),
    compiler_params=pltpu.CompilerParams(
        dimension_semantics=("arbitrary", "arbitrary"),
        vmem_limit_bytes=64 * 1024 * 1024),
)


CCH = 16


def _combine_body(y_hbm, d1_hbm, d2_hbm, p1_hbm, p2_hbm, out_hbm,
                  i1_v, i2_v, p1_v, p2_v, y1_v, y2_v, sem):
    wid = lax.axis_index("s") * NC + lax.axis_index("c")
    base = wid * TPW

    def chunk(j, carry):
        tok = base + j * CCH
        pltpu.sync_copy(d1_hbm.at[pl.ds(tok, CCH)], i1_v)
        pltpu.sync_copy(d2_hbm.at[pl.ds(tok, CCH)], i2_v)
        pltpu.sync_copy(p1_hbm.at[pl.ds(tok, CCH)], p1_v)
        pltpu.sync_copy(p2_hbm.at[pl.ds(tok, CCH)], p2_v)
        cp1 = pltpu.async_copy(y_hbm.at[i1_v], y1_v, sem)
        cp2 = pltpu.async_copy(y_hbm.at[i2_v], y2_v, sem)
        cp1.wait()
        cp2.wait()

        def row(rr, carry2):
            s1 = p1_v[rr]
            s2 = p2_v[rr]
            for cc in range(H // 16):
                a = y1_v[rr, pl.ds(cc * 16, 16)]
                b = y2_v[rr, pl.ds(cc * 16, 16)]
                y1_v[rr, pl.ds(cc * 16, 16)] = s1 * a + s2 * b
            return carry2

        lax.fori_loop(0, CCH, row, 0)
        pltpu.sync_copy(y1_v, out_hbm.at[pl.ds(tok, CCH)])
        return carry

    lax.fori_loop(0, TPW // CCH, chunk, 0)


@functools.cache
def _make_combine():
    return pl.kernel(
        _combine_body,
        mesh=plsc.VectorSubcoreMesh(core_axis_name="c", subcore_axis_name="s"),
        out_type=jax.ShapeDtypeStruct((T, H), jnp.float32),
        scratch_types=[
            pltpu.VMEM((CCH,), jnp.int32),
            pltpu.VMEM((CCH,), jnp.int32),
            pltpu.VMEM((CCH, 16), jnp.float32),
            pltpu.VMEM((CCH, 16), jnp.float32),
            pltpu.VMEM((CCH, H), jnp.float32),
            pltpu.VMEM((CCH, H), jnp.float32),
            pltpu.SemaphoreType.DMA,
        ],
    )


def kernel(hidden_states, Wqkv, W1, W2):
    hid = hidden_states.reshape(T, H)
    dests, probs, meta = _route(hid, Wqkv)
    d1, d2 = dests[0], dests[1]
    p1, p2 = probs[0], probs[1]
    meta_vec = meta[0]
    p1b = jnp.broadcast_to(p1[:, None], (T, 16))
    p2b = jnp.broadcast_to(p2[:, None], (T, 16))
    x_pad = _make_dispatch()(hid, d1, d2)
    y = _gemm(meta_vec, x_pad, W1, W1, W2)
    out = _make_combine()(y, d1, d2, p1b, p2b)
    return out.reshape(B, S, H)


# speedup vs baseline: 2.2445x; 1.0563x over previous
import functools

import jax
import jax.numpy as jnp
from jax import lax
from jax.experimental import pallas as pl
from jax.experimental.pallas import tpu as pltpu
from jax.experimental.pallas import tpu_sc as plsc

B, S, H = 2, 2048, 2048
F = 4096
E = 8
T = B * S
TILE = 512
G = 2
GTILE = G * TILE
NG = 16
P = NG * GTILE
FB = 512
NF = F // FB

NC, NS = 2, 16
NW = NC * NS
TPW = T // NW


def _route_body(hid_ref, wqkv_ref, dests_ref, probs_ref, meta_ref):
    hid = hid_ref[...]
    wqkv = wqkv_ref[...]
    mixed = lax.dot_general(wqkv, hid, (((1,), (1,)), ((), ())),
                            preferred_element_type=jnp.float32)
    q = mixed[0:E, :]
    k = mixed[E:2 * E, :]
    v = mixed[2 * E:3 * E, :]
    cols = []
    for i in range(E):
        s = q[i:i + 1, :] * k
        m = jnp.max(s, axis=0, keepdims=True)
        ex = jnp.exp(s - m)
        attn = ex / jnp.sum(ex, axis=0, keepdims=True)
        cols.append(jnp.sum(attn * v, axis=0, keepdims=True))
    logits = jnp.concatenate(cols, axis=0)

    lane = lax.broadcasted_iota(jnp.int32, (E, T), 0)
    m1 = jnp.max(logits, axis=0, keepdims=True)
    i1 = jnp.min(jnp.where(logits >= m1, lane, E), axis=0, keepdims=True)
    masked = jnp.where(lane == i1, jnp.float32(-1e30), logits)
    m2 = jnp.max(masked, axis=0, keepdims=True)
    i2 = jnp.min(jnp.where(masked >= m2, lane, E), axis=0, keepdims=True)
    r = jnp.exp(m2 - m1)
    p1 = 1.0 / (1.0 + r)
    p2 = r / (1.0 + r)

    oh1 = (lane == i1).astype(jnp.float32)
    oh2 = (lane == i2).astype(jnp.float32)
    c = oh1 + oh2
    incl = c
    d = 1
    while d < T:
        incl = incl + jnp.concatenate(
            [jnp.zeros((E, d), jnp.float32), incl[:, :T - d]], axis=1)
        d *= 2
    excl = (incl - c).astype(jnp.int32)
    counts = incl[:, T - 1:T].astype(jnp.int32)
    st_cnt = (counts + (TILE - 1)) // TILE
    grp_cnt = (counts + (GTILE - 1)) // GTILE
    incl_g = grp_cnt
    d = 1
    while d < E:
        incl_g = incl_g + jnp.concatenate(
            [jnp.zeros((d, 1), jnp.int32), incl_g[:E - d, :]], axis=0)
        d *= 2
    excl_g = incl_g - grp_cnt
    row_off = excl_g * GTILE
    nactg = incl_g[E - 1:E, :]

    dest_base = excl + row_off
    dest1 = jnp.sum(jnp.where(lane == i1, dest_base, 0), axis=0, keepdims=True)
    dest2 = jnp.sum(jnp.where(lane == i2, dest_base, 0), axis=0, keepdims=True)
    dests_ref[0:1, :] = dest1
    dests_ref[1:2, :] = dest2
    probs_ref[0:1, :] = p1
    probs_ref[1:2, :] = p2

    gj = lax.broadcasted_iota(jnp.int32, (E, 64), 1)
    te1 = jnp.minimum(
        jnp.sum((gj >= incl_g).astype(jnp.int32), axis=0, keepdims=True),
        E - 1)
    gj2 = gj - NG
    te2 = jnp.minimum(
        jnp.sum((gj2 >= incl_g).astype(jnp.int32), axis=0, keepdims=True),
        E - 1)
    eidx = lax.broadcasted_iota(jnp.int32, (E, 64), 0)
    oh2 = (eidx == te2).astype(jnp.int32)
    excl_g_j = jnp.sum(oh2 * excl_g, axis=0, keepdims=True)
    grp_j = jnp.sum(oh2 * grp_cnt, axis=0, keepdims=True)
    st_j = jnp.sum(oh2 * st_cnt, axis=0, keepdims=True)
    lane64 = lax.broadcasted_iota(jnp.int32, (1, 64), 1)
    local = (lane64 - NG) - excl_g_j
    stc_j = jnp.where(local < grp_j - 1, G, st_j - G * (grp_j - 1))
    meta_ref[...] = jnp.where(
        lane64 < NG, te1,
        jnp.where(lane64 < 2 * NG, stc_j, nactg))


_route = pl.pallas_call(
    _route_body,
    out_shape=(
        jax.ShapeDtypeStruct((2, T), jnp.int32),
        jax.ShapeDtypeStruct((2, T), jnp.float32),
        jax.ShapeDtypeStruct((1, 64), jnp.int32),
    ),
)


DCH = 32
HW = H // 2


def _dispatch_body(hid_hbm, d1_hbm, d2_hbm, xpad_hbm, i1_v, i2_v, rows_v, sem):
    wid = lax.axis_index("s") * NC + lax.axis_index("c")
    base = wid * TPW

    def body(j, carry):
        tok = base + j * DCH
        ca = pltpu.async_copy(d1_hbm.at[pl.ds(tok, DCH)], i1_v, sem)
        cb = pltpu.async_copy(d2_hbm.at[pl.ds(tok, DCH)], i2_v, sem)
        cc = pltpu.async_copy(hid_hbm.at[pl.ds(tok, DCH)], rows_v, sem)
        ca.wait()
        cb.wait()
        cc.wait()
        c1 = pltpu.async_copy(rows_v, xpad_hbm.at[i1_v], sem)
        c2 = pltpu.async_copy(rows_v, xpad_hbm.at[i2_v], sem)
        c1.wait()
        c2.wait()
        return carry

    lax.fori_loop(0, TPW // DCH, body, 0)


@functools.cache
def _make_dispatch():
    return pl.kernel(
        _dispatch_body,
        mesh=plsc.VectorSubcoreMesh(core_axis_name="c", subcore_axis_name="s"),
        out_type=jax.ShapeDtypeStruct((P, H), jnp.float32),
        scratch_types=[
            pltpu.VMEM((DCH,), jnp.int32),
            pltpu.VMEM((DCH,), jnp.int32),
            pltpu.VMEM((DCH, H), jnp.float32),
            pltpu.SemaphoreType.DMA,
        ],
    )


def _gemm_body(meta_ref, x_ref, w1a_ref, w1b_ref, w2_ref, y_ref):
    g = pl.program_id(0)
    f = pl.program_id(1)
    nactg = meta_ref[2 * NG]
    stc = meta_ref[NG + g]

    @pl.when(g < nactg)
    def _():
        def half(lo):
            x = x_ref[lo:lo + TILE, :]
            a = lax.dot_general(x, w1a_ref[0], (((1,), (1,)), ((), ())),
                                preferred_element_type=jnp.float32)
            bg = lax.dot_general(x, w1b_ref[0], (((1,), (1,)), ((), ())),
                                 preferred_element_type=jnp.float32)
            inter = a * lax.logistic(a) * bg
            y_part = lax.dot_general(inter, w2_ref[0],
                                     (((1,), (1,)), ((), ())),
                                     preferred_element_type=jnp.float32)
            prev = jnp.where(f == 0, jnp.zeros_like(y_part),
                             y_ref[lo:lo + TILE, :])
            y_ref[lo:lo + TILE, :] = prev + y_part

        half(0)

        @pl.when(stc > 1)
        def _():
            half(TILE)


def _x_map(g, f, m):
    return (jnp.minimum(g, m[2 * NG] - 1), 0)


def _w1a_map(g, f, m):
    return (m[g], jnp.where(g < m[2 * NG], f, NF - 1), 0)


def _w1b_map(g, f, m):
    return (m[g], NF + jnp.where(g < m[2 * NG], f, NF - 1), 0)


def _w2_map(g, f, m):
    return (m[g], 0, jnp.where(g < m[2 * NG], f, NF - 1))


_gemm = pl.pallas_call(
    _gemm_body,
    grid_spec=pltpu.PrefetchScalarGridSpec(
        num_scalar_prefetch=1,
        grid=(NG, NF),
        in_specs=[
            pl.BlockSpec((GTILE, H), _x_map),
            pl.BlockSpec((1, FB, H), _w1a_map),
            pl.BlockSpec((1, FB, H), _w1b_map),
            pl.BlockSpec((1, H, FB), _w2_map),
        ],
        out_specs=pl.BlockSpec((GTILE, H), lambda g, f, m: (g, 0)),
    ),
    out_shape=jax.ShapeDtypeStruct((P, H), jnp.float32),
    compiler_params=pltpu.CompilerParams(
        dimension_semantics=("arbitrary", "arbitrary"),
        vmem_limit_bytes=64 * 1024 * 1024),
)


CCH = 16


def _combine_body(y_hbm, d1_hbm, d2_hbm, p1_hbm, p2_hbm, out_hbm,
                  i1_v, i2_v, p1_v, p2_v, y1_v, y2_v, sem):
    wid = lax.axis_index("s") * NC + lax.axis_index("c")
    base = wid * TPW

    def chunk(j, carry):
        tok = base + j * CCH
        ca = pltpu.async_copy(d1_hbm.at[pl.ds(tok, CCH)], i1_v, sem)
        cb = pltpu.async_copy(d2_hbm.at[pl.ds(tok, CCH)], i2_v, sem)
        ca.wait()
        cb.wait()
        cp1 = pltpu.async_copy(y_hbm.at[i1_v], y1_v, sem)
        cp2 = pltpu.async_copy(y_hbm.at[i2_v], y2_v, sem)
        cc = pltpu.async_copy(p1_hbm.at[pl.ds(tok, CCH)], p1_v, sem)
        cd = pltpu.async_copy(p2_hbm.at[pl.ds(tok, CCH)], p2_v, sem)
        cp1.wait()
        cp2.wait()
        cc.wait()
        cd.wait()

        def row(rr, carry2):
            s1 = p1_v[rr]
            s2 = p2_v[rr]
            for cc in range(H // 16):
                a = y1_v[rr, pl.ds(cc * 16, 16)]
                b = y2_v[rr, pl.ds(cc * 16, 16)]
                y1_v[rr, pl.ds(cc * 16, 16)] = s1 * a + s2 * b
            return carry2

        lax.fori_loop(0, CCH, row, 0)
        pltpu.sync_copy(y1_v, out_hbm.at[pl.ds(tok, CCH)])
        return carry

    lax.fori_loop(0, TPW // CCH, chunk, 0)


@functools.cache
def _make_combine():
    return pl.kernel(
        _combine_body,
        mesh=plsc.VectorSubcoreMesh(core_axis_name="c", subcore_axis_name="s"),
        out_type=jax.ShapeDtypeStruct((T, H), jnp.float32),
        scratch_types=[
            pltpu.VMEM((CCH,), jnp.int32),
            pltpu.VMEM((CCH,), jnp.int32),
            pltpu.VMEM((CCH, 16), jnp.float32),
            pltpu.VMEM((CCH, 16), jnp.float32),
            pltpu.VMEM((CCH, H), jnp.float32),
            pltpu.VMEM((CCH, H), jnp.float32),
            pltpu.SemaphoreType.DMA,
        ],
    )


def kernel(hidden_states, Wqkv, W1, W2):
    hid = hidden_states.reshape(T, H)
    dests, probs, meta = _route(hid, Wqkv)
    d1, d2 = dests[0], dests[1]
    p1, p2 = probs[0], probs[1]
    meta_vec = meta[0]
    p1b = jnp.broadcast_to(p1[:, None], (T, 16))
    p2b = jnp.broadcast_to(p2[:, None], (T, 16))
    x_pad = _make_dispatch()(hid, d1, d2)
    y = _gemm(meta_vec, x_pad, W1, W1, W2)
    out = _make_combine()(y, d1, d2, p1b, p2b)
    return out.reshape(B, S, H)


# speedup vs baseline: 2.7874x; 1.2419x over previous
import functools

import jax
import jax.numpy as jnp
from jax import lax
from jax.experimental import pallas as pl
from jax.experimental.pallas import tpu as pltpu
from jax.experimental.pallas import tpu_sc as plsc

B, S, H = 2, 2048, 2048
F = 4096
E = 8
T = B * S
TILE = 576
G = 2
GTILE = G * TILE
NG = 14
P = NG * GTILE
FB = 512
NF = F // FB

NC, NS = 2, 16
NW = NC * NS
TPW = T // NW


def _route_body(hid_ref, wqkv_ref, dests_ref, probs_ref, meta_ref):
    hid = hid_ref[...]
    wqkv = wqkv_ref[...]
    mixed = lax.dot_general(wqkv, hid, (((1,), (1,)), ((), ())),
                            preferred_element_type=jnp.float32)
    q = mixed[0:E, :]
    k = mixed[E:2 * E, :]
    v = mixed[2 * E:3 * E, :]
    cols = []
    for i in range(E):
        s = q[i:i + 1, :] * k
        m = jnp.max(s, axis=0, keepdims=True)
        ex = jnp.exp(s - m)
        attn = ex / jnp.sum(ex, axis=0, keepdims=True)
        cols.append(jnp.sum(attn * v, axis=0, keepdims=True))
    logits = jnp.concatenate(cols, axis=0)

    lane = lax.broadcasted_iota(jnp.int32, (E, T), 0)
    m1 = jnp.max(logits, axis=0, keepdims=True)
    i1 = jnp.min(jnp.where(logits >= m1, lane, E), axis=0, keepdims=True)
    masked = jnp.where(lane == i1, jnp.float32(-1e30), logits)
    m2 = jnp.max(masked, axis=0, keepdims=True)
    i2 = jnp.min(jnp.where(masked >= m2, lane, E), axis=0, keepdims=True)
    r = jnp.exp(m2 - m1)
    p1 = 1.0 / (1.0 + r)
    p2 = r / (1.0 + r)

    oh1 = (lane == i1).astype(jnp.float32)
    oh2 = (lane == i2).astype(jnp.float32)
    c = oh1 + oh2
    incl = c
    d = 1
    while d < T:
        incl = incl + jnp.concatenate(
            [jnp.zeros((E, d), jnp.float32), incl[:, :T - d]], axis=1)
        d *= 2
    excl = (incl - c).astype(jnp.int32)
    counts = incl[:, T - 1:T].astype(jnp.int32)
    st_cnt = (counts + (TILE - 1)) // TILE
    grp_cnt = (counts + (GTILE - 1)) // GTILE
    incl_g = grp_cnt
    d = 1
    while d < E:
        incl_g = incl_g + jnp.concatenate(
            [jnp.zeros((d, 1), jnp.int32), incl_g[:E - d, :]], axis=0)
        d *= 2
    excl_g = incl_g - grp_cnt
    row_off = excl_g * GTILE
    nactg = incl_g[E - 1:E, :]

    dest_base = excl + row_off
    dest1 = jnp.sum(jnp.where(lane == i1, dest_base, 0), axis=0, keepdims=True)
    dest2 = jnp.sum(jnp.where(lane == i2, dest_base, 0), axis=0, keepdims=True)
    dests_ref[0:1, :] = dest1
    dests_ref[1:2, :] = dest2
    probs_ref[0:1, :] = p1
    probs_ref[1:2, :] = p2

    gj = lax.broadcasted_iota(jnp.int32, (E, 64), 1)
    te1 = jnp.minimum(
        jnp.sum((gj >= incl_g).astype(jnp.int32), axis=0, keepdims=True),
        E - 1)
    gj2 = gj - NG
    te2 = jnp.minimum(
        jnp.sum((gj2 >= incl_g).astype(jnp.int32), axis=0, keepdims=True),
        E - 1)
    eidx = lax.broadcasted_iota(jnp.int32, (E, 64), 0)
    oh2 = (eidx == te2).astype(jnp.int32)
    excl_g_j = jnp.sum(oh2 * excl_g, axis=0, keepdims=True)
    grp_j = jnp.sum(oh2 * grp_cnt, axis=0, keepdims=True)
    st_j = jnp.sum(oh2 * st_cnt, axis=0, keepdims=True)
    lane64 = lax.broadcasted_iota(jnp.int32, (1, 64), 1)
    local = (lane64 - NG) - excl_g_j
    stc_j = jnp.where(local < grp_j - 1, G, st_j - G * (grp_j - 1))
    meta_ref[...] = jnp.where(
        lane64 < NG, te1,
        jnp.where(lane64 < 2 * NG, stc_j, nactg))


_route = pl.pallas_call(
    _route_body,
    out_shape=(
        jax.ShapeDtypeStruct((2, T), jnp.int32),
        jax.ShapeDtypeStruct((2, T), jnp.float32),
        jax.ShapeDtypeStruct((1, 64), jnp.int32),
    ),
)


DCH = 32
HW = H // 2


def _dispatch_body(hid_hbm, d1_hbm, d2_hbm, xpad_hbm, i1_v, i2_v, rows_v, sem):
    wid = lax.axis_index("s") * NC + lax.axis_index("c")
    base = wid * TPW

    def body(j, carry):
        tok = base + j * DCH
        ca = pltpu.async_copy(d1_hbm.at[pl.ds(tok, DCH)], i1_v, sem)
        cb = pltpu.async_copy(d2_hbm.at[pl.ds(tok, DCH)], i2_v, sem)
        cc = pltpu.async_copy(hid_hbm.at[pl.ds(tok, DCH)], rows_v, sem)
        ca.wait()
        cb.wait()
        cc.wait()
        c1 = pltpu.async_copy(rows_v, xpad_hbm.at[i1_v], sem)
        c2 = pltpu.async_copy(rows_v, xpad_hbm.at[i2_v], sem)
        c1.wait()
        c2.wait()
        return carry

    lax.fori_loop(0, TPW // DCH, body, 0)


@functools.cache
def _make_dispatch():
    return pl.kernel(
        _dispatch_body,
        mesh=plsc.VectorSubcoreMesh(core_axis_name="c", subcore_axis_name="s"),
        out_type=jax.ShapeDtypeStruct((P, H), jnp.float32),
        scratch_types=[
            pltpu.VMEM((DCH,), jnp.int32),
            pltpu.VMEM((DCH,), jnp.int32),
            pltpu.VMEM((DCH, H), jnp.float32),
            pltpu.SemaphoreType.DMA,
        ],
    )


def _gemm_body(meta_ref, x_ref, w1a_ref, w1b_ref, w2_ref, y_ref):
    g = pl.program_id(0)
    f = pl.program_id(1)
    nactg = meta_ref[2 * NG]
    stc = meta_ref[NG + g]

    @pl.when(g < nactg)
    def _():
        def half(lo):
            x = x_ref[lo:lo + TILE, :]
            a = lax.dot_general(x, w1a_ref[0], (((1,), (1,)), ((), ())),
                                preferred_element_type=jnp.float32)
            bg = lax.dot_general(x, w1b_ref[0], (((1,), (1,)), ((), ())),
                                 preferred_element_type=jnp.float32)
            inter = a * lax.logistic(a) * bg
            y_part = lax.dot_general(inter, w2_ref[0],
                                     (((1,), (1,)), ((), ())),
                                     preferred_element_type=jnp.float32)
            prev = jnp.where(f == 0, jnp.zeros_like(y_part),
                             y_ref[lo:lo + TILE, :])
            y_ref[lo:lo + TILE, :] = prev + y_part

        half(0)

        @pl.when(stc > 1)
        def _():
            half(TILE)


def _x_map(g, f, m):
    return (jnp.minimum(g, m[2 * NG] - 1), 0)


def _w1a_map(g, f, m):
    return (m[g], jnp.where(g < m[2 * NG], f, NF - 1), 0)


def _w1b_map(g, f, m):
    return (m[g], NF + jnp.where(g < m[2 * NG], f, NF - 1), 0)


def _w2_map(g, f, m):
    return (m[g], 0, jnp.where(g < m[2 * NG], f, NF - 1))


_gemm = pl.pallas_call(
    _gemm_body,
    grid_spec=pltpu.PrefetchScalarGridSpec(
        num_scalar_prefetch=1,
        grid=(NG, NF),
        in_specs=[
            pl.BlockSpec((GTILE, H), _x_map),
            pl.BlockSpec((1, FB, H), _w1a_map),
            pl.BlockSpec((1, FB, H), _w1b_map),
            pl.BlockSpec((1, H, FB), _w2_map),
        ],
        out_specs=pl.BlockSpec((GTILE, H), lambda g, f, m: (g, 0)),
    ),
    out_shape=jax.ShapeDtypeStruct((P, H), jnp.float32),
    compiler_params=pltpu.CompilerParams(
        dimension_semantics=("arbitrary", "arbitrary"),
        vmem_limit_bytes=64 * 1024 * 1024),
)


CCH = 16


def _combine_body(y_hbm, d1_hbm, d2_hbm, p1_hbm, p2_hbm, out_hbm,
                  i1_v, i2_v, p1_v, p2_v, y1_v, y2_v, sem):
    wid = lax.axis_index("s") * NC + lax.axis_index("c")
    base = wid * TPW

    def chunk(j, carry):
        tok = base + j * CCH
        ca = pltpu.async_copy(d1_hbm.at[pl.ds(tok, CCH)], i1_v, sem)
        cb = pltpu.async_copy(d2_hbm.at[pl.ds(tok, CCH)], i2_v, sem)
        ca.wait()
        cb.wait()
        cp1 = pltpu.async_copy(y_hbm.at[i1_v], y1_v, sem)
        cp2 = pltpu.async_copy(y_hbm.at[i2_v], y2_v, sem)
        cc = pltpu.async_copy(p1_hbm.at[pl.ds(tok, CCH)], p1_v, sem)
        cd = pltpu.async_copy(p2_hbm.at[pl.ds(tok, CCH)], p2_v, sem)
        cp1.wait()
        cp2.wait()
        cc.wait()
        cd.wait()

        def row(rr, carry2):
            s1 = p1_v[rr]
            s2 = p2_v[rr]
            for cc in range(H // 16):
                a = y1_v[rr, pl.ds(cc * 16, 16)]
                b = y2_v[rr, pl.ds(cc * 16, 16)]
                y1_v[rr, pl.ds(cc * 16, 16)] = s1 * a + s2 * b
            return carry2

        lax.fori_loop(0, CCH, row, 0)
        pltpu.sync_copy(y1_v, out_hbm.at[pl.ds(tok, CCH)])
        return carry

    lax.fori_loop(0, TPW // CCH, chunk, 0)


@functools.cache
def _make_combine():
    return pl.kernel(
        _combine_body,
        mesh=plsc.VectorSubcoreMesh(core_axis_name="c", subcore_axis_name="s"),
        out_type=jax.ShapeDtypeStruct((T, H), jnp.float32),
        scratch_types=[
            pltpu.VMEM((CCH,), jnp.int32),
            pltpu.VMEM((CCH,), jnp.int32),
            pltpu.VMEM((CCH, 16), jnp.float32),
            pltpu.VMEM((CCH, 16), jnp.float32),
            pltpu.VMEM((CCH, H), jnp.float32),
            pltpu.VMEM((CCH, H), jnp.float32),
            pltpu.SemaphoreType.DMA,
        ],
    )


def kernel(hidden_states, Wqkv, W1, W2):
    hid = hidden_states.reshape(T, H)
    dests, probs, meta = _route(hid, Wqkv)
    d1, d2 = dests[0], dests[1]
    p1, p2 = probs[0], probs[1]
    meta_vec = meta[0]
    p1b = jnp.broadcast_to(p1[:, None], (T, 16))
    p2b = jnp.broadcast_to(p2[:, None], (T, 16))
    x_pad = _make_dispatch()(hid, d1, d2)
    y = _gemm(meta_vec, x_pad, W1, W1, W2)
    out = _make_combine()(y, d1, d2, p1b, p2b)
    return out.reshape(B, S, H)
